# baseline (speedup 1.0000x reference)
import dataclasses
import functools

import jax
import jax.numpy as jnp
from jax import lax
from jax.experimental import pallas as pl
from jax.experimental.pallas import tpu as pltpu
from jax.experimental.pallas import tpu_sc as plsc

N_PROT = 10000
N_DIS = 10000
E_PP = 320000
E_PD = 160000
E_SUP = 100000
D = 128

NC = 2
NS = 16
NW = NC * NS
CHA = 112
CHD = 128
RB = 80
TRASH = 400
ACC_N = N_PROT + TRASH

T_PP = 91
T_PD = 46
T_SUP = 26

_mesh = plsc.VectorSubcoreMesh(core_axis_name="c", subcore_axis_name="s",
                               num_cores=NC, num_subcores=NS)

_sc_params = pltpu.CompilerParams()
if "needs_layout_passes" in pltpu.CompilerParams.__dataclass_fields__:
    _sc_params = dataclasses.replace(_sc_params, needs_layout_passes=False)
_sc_params_untiled = dataclasses.replace(_sc_params, use_tc_tiling_on_sc=False)


def _pad_agg(src, dst, w, t_chunks):
    e_pad = NW * t_chunks * CHA
    n = src.shape[0]
    ar = jnp.arange(e_pad - n, dtype=jnp.int32)
    srcp = jnp.concatenate([src, ar % N_PROT])
    dstp = jnp.concatenate([dst, N_PROT + ar % TRASH])
    if w is None:
        return srcp, dstp, None
    wp = jnp.concatenate([w, jnp.zeros((e_pad - n,), jnp.float32)])
    return srcp, dstp, wp


def _pack_dec(src, dst, t_chunks):
    e_pad = NW * t_chunks * CHD
    n = src.shape[0]
    ar = jnp.arange(e_pad - n, dtype=jnp.int32)
    srcp = jnp.concatenate([src, ar % N_PROT])
    dstp = jnp.concatenate([dst + N_PROT, N_PROT + ar % N_DIS])
    return jnp.stack([srcp.reshape(NW, t_chunks, CHD),
                      dstp.reshape(NW, t_chunks, CHD)], axis=2)


def _fill_zeros(buf_v):
    z16 = jnp.zeros((16,), jnp.float32)

    @pl.loop(0, RB)
    def _(i):
        for j in range(D // 16):
            buf_v[i, pl.ds(j * 16, 16)] = z16


def _zero_acc(buf_v, acc_sh, s):
    @pl.loop(s, N_PROT // RB, step=NS)
    def _(t):
        pltpu.sync_copy(buf_v.at[pl.ds(0, RB)], acc_sh.at[pl.ds(t * RB, RB)])


def _write_acc(acc_sh, out_hbm, c, s):
    @pl.loop(s, N_PROT // RB, step=NS)
    def _(t):
        pltpu.sync_copy(acc_sh.at[pl.ds(t * RB, RB)],
                        out_hbm.at[c].at[pl.ds(t * RB, RB)])


def _seg_phase(xp_hbm, src_hbm, dst_hbm, w_hbm, acc_sh, wid,
               sb, db, wb, rows, ssb, sdb, sg, ss, t_chunks, weighted):
    base0 = wid * t_chunks * CHA

    def issue_sb(t, b):
        pltpu.async_copy(src_hbm.at[pl.ds(base0 + t * CHA, CHA)], sb[b],
                         ssb[b])
        if weighted:
            pltpu.async_copy(w_hbm.at[pl.ds(base0 + t * CHA, CHA)], wb[b],
                             ssb[b])

    def wait_sb(t, b):
        pltpu.make_async_copy(src_hbm.at[pl.ds(base0 + t * CHA, CHA)], sb[b],
                              ssb[b]).wait()
        if weighted:
            pltpu.make_async_copy(w_hbm.at[pl.ds(base0 + t * CHA, CHA)],
                                  wb[b], ssb[b]).wait()

    def issue_db(t, b):
        pltpu.async_copy(dst_hbm.at[pl.ds(base0 + t * CHA, CHA)], db[b],
                         sdb[b])

    def wait_db(t, b):
        pltpu.make_async_copy(dst_hbm.at[pl.ds(base0 + t * CHA, CHA)], db[b],
                              sdb[b]).wait()

    def issue_g(t, b):
        pltpu.async_copy(xp_hbm.at[sb[b]], rows[b], sg[b])

    def wait_g(t, b):
        pltpu.make_async_copy(xp_hbm.at[sb[b]], rows[b], sg[b]).wait()

    def issue_s(t, b):
        pltpu.async_copy(rows[b], acc_sh.at[db[b]], ss[b], add=True)

    def wait_s(t, b):
        pltpu.make_async_copy(rows[b], acc_sh.at[db[b]], ss[b]).wait()

    def mult(t, b):
        if weighted:
            @plsc.parallel_loop(0, CHA, unroll=4)
            def _(e):
                wv = plsc.load_gather(wb[b], [jnp.full((16,), e, jnp.int32)])
                rbuf = rows[b]
                for j in range(D // 16):
                    sl = pl.ds(j * 16, 16)
                    rbuf[e, sl] = rbuf[e, sl] * wv

    def body(c, b, do_ws, do_next, do_sb2):
        b1, b2 = (b + 1) % 3, (b + 2) % 3
        if do_ws:
            wait_s(c - 2, b1)
        if do_next:
            issue_db(c + 1, b1)
            wait_sb(c + 1, b1)
            issue_g(c + 1, b1)
        wait_g(c, b)
        if do_sb2:
            issue_sb(c + 2, b2)
        mult(c, b)
        wait_db(c, b)
        issue_s(c, b)

    issue_sb(0, 0)
    issue_sb(1, 1)
    issue_db(0, 0)
    wait_sb(0, 0)
    issue_g(0, 0)

    body(0, 0, False, True, True)
    body(1, 1, False, True, True)

    @pl.loop(0, (t_chunks - 4) // 3)
    def _(tau):
        for i in range(3):
            c = 2 + tau * 3 + i
            body(c, (2 + i) % 3, True, True, True)

    body(t_chunks - 2, (t_chunks - 2) % 3, True, True, False)
    body(t_chunks - 1, (t_chunks - 1) % 3, True, False, False)
    wait_s(t_chunks - 2, (t_chunks - 2) % 3)
    wait_s(t_chunks - 1, (t_chunks - 1) % 3)


def _agg_body(xp_hbm, srcpp_hbm, dstpp_hbm, wpp_hbm, srcpd_hbm, dstpd_hbm,
              aggpp_hbm, aggpd_hbm,
              sb0, sb1, sb2, db0, db1, db2, wb0, wb1, wb2,
              rows0, rows1, rows2, acc_sh,
              ssb0, ssb1, ssb2, sdb0, sdb1, sdb2,
              sg0, sg1, sg2, ss0, ss1, ss2):
    c = lax.axis_index("c")
    s = lax.axis_index("s")
    wid = s * NC + c
    sb = [sb0, sb1, sb2]
    db = [db0, db1, db2]
    wb = [wb0, wb1, wb2]
    rows = [rows0, rows1, rows2]
    ssb = [ssb0, ssb1, ssb2]
    sdb = [sdb0, sdb1, sdb2]
    sg = [sg0, sg1, sg2]
    ss = [ss0, ss1, ss2]

    _fill_zeros(rows0)
    _zero_acc(rows0, acc_sh, s)
    plsc.subcore_barrier()

    _seg_phase(xp_hbm, srcpp_hbm, dstpp_hbm, wpp_hbm, acc_sh, wid,
               sb, db, wb, rows, ssb, sdb, sg, ss, T_PP, True)
    plsc.subcore_barrier()
    _write_acc(acc_sh, aggpp_hbm, c, s)
    _fill_zeros(rows0)
    _zero_acc(rows0, acc_sh, s)
    plsc.subcore_barrier()

    _seg_phase(xp_hbm, srcpd_hbm, dstpd_hbm, None, acc_sh, wid,
               sb, db, wb, rows, ssb, sdb, sg, ss, T_PD, False)
    plsc.subcore_barrier()
    _write_acc(acc_sh, aggpd_hbm, c, s)


@jax.jit
def _aggregate(xp, srcpp, dstpp, wpp, srcpd, dstpd):
    f = pl.kernel(
        _agg_body,
        out_type=(jax.ShapeDtypeStruct((NC, N_PROT, D), jnp.float32),
                  jax.ShapeDtypeStruct((NC, N_DIS, D), jnp.float32)),
        mesh=_mesh,
        scratch_types=[
            pltpu.VMEM((CHA,), jnp.int32),
            pltpu.VMEM((CHA,), jnp.int32),
            pltpu.VMEM((CHA,), jnp.int32),
            pltpu.VMEM((CHA,), jnp.int32),
            pltpu.VMEM((CHA,), jnp.int32),
            pltpu.VMEM((CHA,), jnp.int32),
            pltpu.VMEM((CHA,), jnp.float32),
            pltpu.VMEM((CHA,), jnp.float32),
            pltpu.VMEM((CHA,), jnp.float32),
            pltpu.VMEM((CHA, D), jnp.float32),
            pltpu.VMEM((CHA, D), jnp.float32),
            pltpu.VMEM((CHA, D), jnp.float32),
            pltpu.VMEM_SHARED((ACC_N, D), jnp.float32),
            pltpu.SemaphoreType.DMA,
            pltpu.SemaphoreType.DMA,
            pltpu.SemaphoreType.DMA,
            pltpu.SemaphoreType.DMA,
            pltpu.SemaphoreType.DMA,
            pltpu.SemaphoreType.DMA,
            pltpu.SemaphoreType.DMA,
            pltpu.SemaphoreType.DMA,
            pltpu.SemaphoreType.DMA,
            pltpu.SemaphoreType.DMA,
            pltpu.SemaphoreType.DMA,
            pltpu.SemaphoreType.DMA,
        ],
        compiler_params=_sc_params,
    )
    return f(xp, srcpp, dstpp, wpp, srcpd, dstpd)


def _enc_block(x_ref, a_ref, ws_ref, wn_ref, o_ref):
    agg = a_ref[0] + a_ref[1]
    h = jnp.maximum(
        jnp.dot(x_ref[...], ws_ref[0], preferred_element_type=jnp.float32)
        + jnp.dot(agg, wn_ref[0], preferred_element_type=jnp.float32),
        0.0)
    hb = h.astype(jnp.bfloat16)
    lo = lax.bitcast_convert_type(hb[:, :D // 2], jnp.uint16)
    hi = lax.bitcast_convert_type(hb[:, D // 2:], jnp.uint16)
    packed = (lo.astype(jnp.uint32)
              | (hi.astype(jnp.uint32) << 16)).astype(jnp.int32)
    o_ref[...] = packed


@jax.jit
def _encode(x_p, x_d, aggpp2, aggpd2, w_self_p, w_nbr_pp, w_self_d,
            w_nbr_pd):
    x_all = jnp.concatenate([x_p, x_d])
    agg_all = jnp.concatenate([aggpp2, aggpd2], axis=1)
    ws = jnp.stack([w_self_p, w_self_d])
    wn = jnp.stack([w_nbr_pp, w_nbr_pd])
    n = x_all.shape[0]
    br = 2000
    nb = n // br
    return pl.pallas_call(
        _enc_block,
        grid=(nb,),
        in_specs=[
            pl.BlockSpec((br, D), lambda i: (i, 0)),
            pl.BlockSpec((NC, br, D), lambda i: (0, i, 0)),
            pl.BlockSpec((1, D, D), lambda i: (i // (nb // 2), 0, 0)),
            pl.BlockSpec((1, D, D), lambda i: (i // (nb // 2), 0, 0)),
        ],
        out_specs=pl.BlockSpec((br, D // 2), lambda i: (i, 0)),
        out_shape=jax.ShapeDtypeStruct((n, D // 2), jnp.int32),
    )(x_all, agg_all, ws, wn)


def _decode_body(h_hbm, sup_hbm, out_hbm,
                 idx_v, l0, l1, l2, r0, r1, r2, oall_v,
                 semi, sl0, sl1, sl2, sr0, sr1, sr2):
    c = lax.axis_index("c")
    s = lax.axis_index("s")
    wid = s * NC + c
    lbufs = [l0, l1, l2]
    rbufs = [r0, r1, r2]
    sls = [sl0, sl1, sl2]
    srs = [sr0, sr1, sr2]
    lane = lax.iota(jnp.int32, 16)

    pltpu.async_copy(sup_hbm.at[wid], idx_v, semi).wait()

    def issue(t, b):
        pltpu.async_copy(h_hbm.at[idx_v.at[t, 0]], lbufs[b], sls[b])
        pltpu.async_copy(h_hbm.at[idx_v.at[t, 1]], rbufs[b], srs[b])

    def wait(t, b):
        pltpu.make_async_copy(h_hbm.at[idx_v.at[t, 0]], lbufs[b],
                              sls[b]).wait()
        pltpu.make_async_copy(h_hbm.at[idx_v.at[t, 1]], rbufs[b],
                              srs[b]).wait()

    def compute(t, b):
        l_v, r_v = lbufs[b], rbufs[b]

        @plsc.parallel_loop(0, CHD // 16, unroll=2)
        def _(g):
            out16 = jnp.zeros((16,), jnp.float32)
            for r in range(16):
                e = g * 16 + r
                acc = None
                for j in range(D // 32):
                    sl = pl.ds(j * 16, 16)
                    lv = plsc.bitcast(l_v[e, sl], jnp.bfloat16)
                    rv = plsc.bitcast(r_v[e, sl], jnp.bfloat16)
                    prod = lv * rv
                    acc = prod if acc is None else acc + prod
                pa, pb = plsc.unpack(acc, format=plsc.PackFormat.INTERLEAVED)
                dot = jnp.sum(pa + pb)
                out16 = jnp.where(lane == r, dot, out16)
            oall_v[t, pl.ds(g * 16, 16)] = out16

    issue(0, 0)
    issue(1, 1)

    @pl.loop(0, (T_SUP - 2) // 3)
    def _(tau):
        for i in range(3):
            t = tau * 3 + i
            issue(t + 2, (i + 2) % 3)
            wait(t, i)
            compute(t, i)

    for t in (T_SUP - 2, T_SUP - 1):
        wait(t, t % 3)
        compute(t, t % 3)

    pltpu.sync_copy(oall_v, out_hbm.at[wid])


@jax.jit
def _decode(h_all, sup_pack):
    f = pl.kernel(
        _decode_body,
        out_type=jax.ShapeDtypeStruct((NW, T_SUP, CHD), jnp.float32),
        mesh=_mesh,
        scratch_types=[
            pltpu.VMEM((T_SUP, 2, CHD), jnp.int32),
            pltpu.VMEM((CHD, D // 2), jnp.int32),
            pltpu.VMEM((CHD, D // 2), jnp.int32),
            pltpu.VMEM((CHD, D // 2), jnp.int32),
            pltpu.VMEM((CHD, D // 2), jnp.int32),
            pltpu.VMEM((CHD, D // 2), jnp.int32),
            pltpu.VMEM((CHD, D // 2), jnp.int32),
            pltpu.VMEM((T_SUP, CHD), jnp.float32),
            pltpu.SemaphoreType.DMA,
            pltpu.SemaphoreType.DMA,
            pltpu.SemaphoreType.DMA,
            pltpu.SemaphoreType.DMA,
            pltpu.SemaphoreType.DMA,
            pltpu.SemaphoreType.DMA,
            pltpu.SemaphoreType.DMA,
        ],
        compiler_params=_sc_params_untiled,
    )
    return f(h_all, sup_pack)


def kernel(x_protein, x_disease, edge_index_pp, edge_attr_pp, edge_index_pd,
           sup_edge_index, W_self_p, W_nbr_pp, W_self_d, W_nbr_pd):
    srcpp, dstpp, wpp = _pad_agg(edge_index_pp[0], edge_index_pp[1],
                                 edge_attr_pp[:, 0], T_PP)
    srcpd, dstpd, _ = _pad_agg(edge_index_pd[0], edge_index_pd[1], None,
                               T_PD)
    sup_pack = _pack_dec(sup_edge_index[0], sup_edge_index[1], T_SUP)

    aggpp2, aggpd2 = _aggregate(x_protein, srcpp, dstpp, wpp, srcpd, dstpd)
    h_all = _encode(x_protein, x_disease, aggpp2, aggpd2,
                    W_self_p, W_nbr_pp, W_self_d, W_nbr_pd)
    scores = _decode(h_all, sup_pack)
    return scores.reshape(-1)[:E_SUP]


# speedup vs baseline: 1.0424x; 1.0424x over previous
import dataclasses
import functools

import jax
import jax.numpy as jnp
from jax import lax
from jax.experimental import pallas as pl
from jax.experimental.pallas import tpu as pltpu
from jax.experimental.pallas import tpu_sc as plsc

N_PROT = 10000
N_DIS = 10000
E_PP = 320000
E_PD = 160000
E_SUP = 100000
D = 128

NC = 2
NS = 16
NW = NC * NS
CHA = 112
CHD = 128
RB = 80
TRASH = 400
ACC_N = N_PROT + TRASH

T_PP = 91
T_PD = 46
T_SUP = 26

_mesh = plsc.VectorSubcoreMesh(core_axis_name="c", subcore_axis_name="s",
                               num_cores=NC, num_subcores=NS)

_sc_params = pltpu.CompilerParams()
if "needs_layout_passes" in pltpu.CompilerParams.__dataclass_fields__:
    _sc_params = dataclasses.replace(_sc_params, needs_layout_passes=False)
_sc_params_untiled = dataclasses.replace(_sc_params, use_tc_tiling_on_sc=False)


def _pad_agg(src, dst, w, t_chunks):
    e_pad = NW * t_chunks * CHA
    n = src.shape[0]
    ar = jnp.arange(e_pad - n, dtype=jnp.int32)
    srcp = jnp.concatenate([src, ar % N_PROT])
    dstp = jnp.concatenate([dst, N_PROT + ar % TRASH])
    if w is None:
        return srcp, dstp, None
    wp = jnp.concatenate([w, jnp.zeros((e_pad - n,), jnp.float32)])
    return srcp, dstp, wp


def _pack_dec(src, dst, t_chunks):
    e_pad = NW * t_chunks * CHD
    n = src.shape[0]
    ar = jnp.arange(e_pad - n, dtype=jnp.int32)
    srcp = jnp.concatenate([src, ar % N_PROT])
    dstp = jnp.concatenate([dst, ar % N_DIS])
    return jnp.stack([srcp.reshape(NW, t_chunks, CHD),
                      dstp.reshape(NW, t_chunks, CHD)], axis=2)


def _fill_zeros(buf_v):
    z16 = jnp.zeros((16,), jnp.float32)

    @pl.loop(0, RB)
    def _(i):
        for j in range(D // 16):
            buf_v[i, pl.ds(j * 16, 16)] = z16


def _zero_acc(buf_v, acc_sh, s):
    @pl.loop(s, N_PROT // RB, step=NS)
    def _(t):
        pltpu.sync_copy(buf_v.at[pl.ds(0, RB)], acc_sh.at[pl.ds(t * RB, RB)])


def _write_acc(acc_sh, out_hbm, c, s):
    @pl.loop(s, N_PROT // RB, step=NS)
    def _(t):
        pltpu.sync_copy(acc_sh.at[pl.ds(t * RB, RB)],
                        out_hbm.at[c].at[pl.ds(t * RB, RB)])


def _seg_phase(xp_hbm, src_hbm, dst_hbm, w_hbm, acc_sh, wid,
               sb, db, wb, rows, ssb, sdb, sg, ss, t_chunks, weighted):
    base0 = wid * t_chunks * CHA

    def issue_sb(t, b):
        pltpu.async_copy(src_hbm.at[pl.ds(base0 + t * CHA, CHA)], sb[b],
                         ssb[b])
        if weighted:
            pltpu.async_copy(w_hbm.at[pl.ds(base0 + t * CHA, CHA)], wb[b],
                             ssb[b])

    def wait_sb(t, b):
        pltpu.make_async_copy(src_hbm.at[pl.ds(base0 + t * CHA, CHA)], sb[b],
                              ssb[b]).wait()
        if weighted:
            pltpu.make_async_copy(w_hbm.at[pl.ds(base0 + t * CHA, CHA)],
                                  wb[b], ssb[b]).wait()

    def issue_db(t, b):
        pltpu.async_copy(dst_hbm.at[pl.ds(base0 + t * CHA, CHA)], db[b],
                         sdb[b])

    def wait_db(t, b):
        pltpu.make_async_copy(dst_hbm.at[pl.ds(base0 + t * CHA, CHA)], db[b],
                              sdb[b]).wait()

    def issue_g(t, b):
        pltpu.async_copy(xp_hbm.at[sb[b]], rows[b], sg[b])

    def wait_g(t, b):
        pltpu.make_async_copy(xp_hbm.at[sb[b]], rows[b], sg[b]).wait()

    def issue_s(t, b):
        pltpu.async_copy(rows[b], acc_sh.at[db[b]], ss[b], add=True)

    def wait_s(t, b):
        pltpu.make_async_copy(rows[b], acc_sh.at[db[b]], ss[b]).wait()

    def mult(t, b):
        if weighted:
            @plsc.parallel_loop(0, CHA, unroll=4)
            def _(e):
                wv = plsc.load_gather(wb[b], [jnp.full((16,), e, jnp.int32)])
                rbuf = rows[b]
                for j in range(D // 16):
                    sl = pl.ds(j * 16, 16)
                    rbuf[e, sl] = rbuf[e, sl] * wv

    def body(c, b, do_ws, do_next, do_sb2):
        b1, b2 = (b + 1) % 3, (b + 2) % 3
        if do_ws:
            wait_s(c - 2, b1)
        if do_next:
            issue_db(c + 1, b1)
            wait_sb(c + 1, b1)
            issue_g(c + 1, b1)
        wait_g(c, b)
        if do_sb2:
            issue_sb(c + 2, b2)
        mult(c, b)
        wait_db(c, b)
        issue_s(c, b)

    issue_sb(0, 0)
    issue_sb(1, 1)
    issue_db(0, 0)
    wait_sb(0, 0)
    issue_g(0, 0)

    body(0, 0, False, True, True)
    body(1, 1, False, True, True)

    @pl.loop(0, (t_chunks - 4) // 3)
    def _(tau):
        for i in range(3):
            c = 2 + tau * 3 + i
            body(c, (2 + i) % 3, True, True, True)

    body(t_chunks - 2, (t_chunks - 2) % 3, True, True, False)
    body(t_chunks - 1, (t_chunks - 1) % 3, True, False, False)
    wait_s(t_chunks - 2, (t_chunks - 2) % 3)
    wait_s(t_chunks - 1, (t_chunks - 1) % 3)


def _agg_body(xp_hbm, srcpp_hbm, dstpp_hbm, wpp_hbm, srcpd_hbm, dstpd_hbm,
              aggpp_hbm, aggpd_hbm,
              sb0, sb1, sb2, db0, db1, db2, wb0, wb1, wb2,
              rows0, rows1, rows2, acc_sh,
              ssb0, ssb1, ssb2, sdb0, sdb1, sdb2,
              sg0, sg1, sg2, ss0, ss1, ss2):
    c = lax.axis_index("c")
    s = lax.axis_index("s")
    wid = s * NC + c
    sb = [sb0, sb1, sb2]
    db = [db0, db1, db2]
    wb = [wb0, wb1, wb2]
    rows = [rows0, rows1, rows2]
    ssb = [ssb0, ssb1, ssb2]
    sdb = [sdb0, sdb1, sdb2]
    sg = [sg0, sg1, sg2]
    ss = [ss0, ss1, ss2]

    _fill_zeros(rows0)
    _zero_acc(rows0, acc_sh, s)
    plsc.subcore_barrier()

    _seg_phase(xp_hbm, srcpp_hbm, dstpp_hbm, wpp_hbm, acc_sh, wid,
               sb, db, wb, rows, ssb, sdb, sg, ss, T_PP, True)
    plsc.subcore_barrier()
    _write_acc(acc_sh, aggpp_hbm, c, s)
    _fill_zeros(rows0)
    _zero_acc(rows0, acc_sh, s)
    plsc.subcore_barrier()

    _seg_phase(xp_hbm, srcpd_hbm, dstpd_hbm, None, acc_sh, wid,
               sb, db, wb, rows, ssb, sdb, sg, ss, T_PD, False)
    plsc.subcore_barrier()
    _write_acc(acc_sh, aggpd_hbm, c, s)


@jax.jit
def _aggregate(xp, srcpp, dstpp, wpp, srcpd, dstpd):
    f = pl.kernel(
        _agg_body,
        out_type=(jax.ShapeDtypeStruct((NC, N_PROT, D), jnp.float32),
                  jax.ShapeDtypeStruct((NC, N_DIS, D), jnp.float32)),
        mesh=_mesh,
        scratch_types=[
            pltpu.VMEM((CHA,), jnp.int32),
            pltpu.VMEM((CHA,), jnp.int32),
            pltpu.VMEM((CHA,), jnp.int32),
            pltpu.VMEM((CHA,), jnp.int32),
            pltpu.VMEM((CHA,), jnp.int32),
            pltpu.VMEM((CHA,), jnp.int32),
            pltpu.VMEM((CHA,), jnp.float32),
            pltpu.VMEM((CHA,), jnp.float32),
            pltpu.VMEM((CHA,), jnp.float32),
            pltpu.VMEM((CHA, D), jnp.float32),
            pltpu.VMEM((CHA, D), jnp.float32),
            pltpu.VMEM((CHA, D), jnp.float32),
            pltpu.VMEM_SHARED((ACC_N, D), jnp.float32),
            pltpu.SemaphoreType.DMA,
            pltpu.SemaphoreType.DMA,
            pltpu.SemaphoreType.DMA,
            pltpu.SemaphoreType.DMA,
            pltpu.SemaphoreType.DMA,
            pltpu.SemaphoreType.DMA,
            pltpu.SemaphoreType.DMA,
            pltpu.SemaphoreType.DMA,
            pltpu.SemaphoreType.DMA,
            pltpu.SemaphoreType.DMA,
            pltpu.SemaphoreType.DMA,
            pltpu.SemaphoreType.DMA,
        ],
        compiler_params=_sc_params,
    )
    return f(xp, srcpp, dstpp, wpp, srcpd, dstpd)


def _enc_block(x_ref, a_ref, ws_ref, wn_ref, o_ref):
    agg = a_ref[0] + a_ref[1]
    h = jnp.maximum(
        jnp.dot(x_ref[...], ws_ref[0], preferred_element_type=jnp.float32)
        + jnp.dot(agg, wn_ref[0], preferred_element_type=jnp.float32),
        0.0)
    hb = h.astype(jnp.bfloat16)
    lo = lax.bitcast_convert_type(hb[:, :D // 2], jnp.uint16)
    hi = lax.bitcast_convert_type(hb[:, D // 2:], jnp.uint16)
    packed = (lo.astype(jnp.uint32)
              | (hi.astype(jnp.uint32) << 16)).astype(jnp.int32)
    o_ref[...] = packed


@jax.jit
def _encode(x, agg2, w_self, w_nbr):
    n = x.shape[0]
    br = 2000
    return pl.pallas_call(
        _enc_block,
        grid=(n // br,),
        in_specs=[
            pl.BlockSpec((br, D), lambda i: (i, 0)),
            pl.BlockSpec((NC, br, D), lambda i: (0, i, 0)),
            pl.BlockSpec((1, D, D), lambda i: (0, 0, 0)),
            pl.BlockSpec((1, D, D), lambda i: (0, 0, 0)),
        ],
        out_specs=pl.BlockSpec((br, D // 2), lambda i: (i, 0)),
        out_shape=jax.ShapeDtypeStruct((n, D // 2), jnp.int32),
    )(x, agg2, w_self[None], w_nbr[None])


def _decode_body(hp_hbm, hd_hbm, sup_hbm, out_hbm,
                 idx_v, l0, l1, l2, r0, r1, r2, oall_v,
                 semi, sl0, sl1, sl2, sr0, sr1, sr2):
    c = lax.axis_index("c")
    s = lax.axis_index("s")
    wid = s * NC + c
    lbufs = [l0, l1, l2]
    rbufs = [r0, r1, r2]
    sls = [sl0, sl1, sl2]
    srs = [sr0, sr1, sr2]
    lane = lax.iota(jnp.int32, 16)

    pltpu.async_copy(sup_hbm.at[wid], idx_v, semi).wait()

    def issue(t, b):
        pltpu.async_copy(hp_hbm.at[idx_v.at[t, 0]], lbufs[b], sls[b])
        pltpu.async_copy(hd_hbm.at[idx_v.at[t, 1]], rbufs[b], srs[b])

    def wait(t, b):
        pltpu.make_async_copy(hp_hbm.at[idx_v.at[t, 0]], lbufs[b],
                              sls[b]).wait()
        pltpu.make_async_copy(hd_hbm.at[idx_v.at[t, 1]], rbufs[b],
                              srs[b]).wait()

    def compute(t, b):
        l_v, r_v = lbufs[b], rbufs[b]

        @plsc.parallel_loop(0, CHD // 16, unroll=2)
        def _(g):
            out16 = jnp.zeros((16,), jnp.float32)
            for r in range(16):
                e = g * 16 + r
                acc = None
                for j in range(D // 32):
                    sl = pl.ds(j * 16, 16)
                    lv = plsc.bitcast(l_v[e, sl], jnp.bfloat16)
                    rv = plsc.bitcast(r_v[e, sl], jnp.bfloat16)
                    prod = lv * rv
                    acc = prod if acc is None else acc + prod
                pa, pb = plsc.unpack(acc, format=plsc.PackFormat.INTERLEAVED)
                dot = jnp.sum(pa + pb)
                out16 = jnp.where(lane == r, dot, out16)
            oall_v[t, pl.ds(g * 16, 16)] = out16

    issue(0, 0)
    issue(1, 1)

    @pl.loop(0, (T_SUP - 2) // 3)
    def _(tau):
        for i in range(3):
            t = tau * 3 + i
            issue(t + 2, (i + 2) % 3)
            wait(t, i)
            compute(t, i)

    for t in (T_SUP - 2, T_SUP - 1):
        wait(t, t % 3)
        compute(t, t % 3)

    pltpu.sync_copy(oall_v, out_hbm.at[wid])


@jax.jit
def _decode(hp, hd, sup_pack):
    f = pl.kernel(
        _decode_body,
        out_type=jax.ShapeDtypeStruct((NW, T_SUP, CHD), jnp.float32),
        mesh=_mesh,
        scratch_types=[
            pltpu.VMEM((T_SUP, 2, CHD), jnp.int32),
            pltpu.VMEM((CHD, D // 2), jnp.int32),
            pltpu.VMEM((CHD, D // 2), jnp.int32),
            pltpu.VMEM((CHD, D // 2), jnp.int32),
            pltpu.VMEM((CHD, D // 2), jnp.int32),
            pltpu.VMEM((CHD, D // 2), jnp.int32),
            pltpu.VMEM((CHD, D // 2), jnp.int32),
            pltpu.VMEM((T_SUP, CHD), jnp.float32),
            pltpu.SemaphoreType.DMA,
            pltpu.SemaphoreType.DMA,
            pltpu.SemaphoreType.DMA,
            pltpu.SemaphoreType.DMA,
            pltpu.SemaphoreType.DMA,
            pltpu.SemaphoreType.DMA,
            pltpu.SemaphoreType.DMA,
        ],
        compiler_params=_sc_params_untiled,
    )
    return f(hp, hd, sup_pack)


def kernel(x_protein, x_disease, edge_index_pp, edge_attr_pp, edge_index_pd,
           sup_edge_index, W_self_p, W_nbr_pp, W_self_d, W_nbr_pd):
    srcpp, dstpp, wpp = _pad_agg(edge_index_pp[0], edge_index_pp[1],
                                 edge_attr_pp[:, 0], T_PP)
    srcpd, dstpd, _ = _pad_agg(edge_index_pd[0], edge_index_pd[1], None,
                               T_PD)
    sup_pack = _pack_dec(sup_edge_index[0], sup_edge_index[1], T_SUP)

    aggpp2, aggpd2 = _aggregate(x_protein, srcpp, dstpp, wpp, srcpd, dstpd)
    hp32 = _encode(x_protein, aggpp2, W_self_p, W_nbr_pp)
    hd32 = _encode(x_disease, aggpd2, W_self_d, W_nbr_pd)
    scores = _decode(hp32, hd32, sup_pack)
    return scores.reshape(-1)[:E_SUP]


# speedup vs baseline: 1.0438x; 1.0013x over previous
import dataclasses
import functools

import jax
import jax.numpy as jnp
from jax import lax
from jax.experimental import pallas as pl
from jax.experimental.pallas import tpu as pltpu
from jax.experimental.pallas import tpu_sc as plsc

N_PROT = 10000
N_DIS = 10000
E_PP = 320000
E_PD = 160000
E_SUP = 100000
D = 128

NC = 2
NS = 16
NW = NC * NS
CHA = 120
CHD = 128
RB = 80
TRASH = 336
ACC_N = N_PROT + TRASH

T_PP = 85
T_PD = 43
T_SUP = 26

_mesh = plsc.VectorSubcoreMesh(core_axis_name="c", subcore_axis_name="s",
                               num_cores=NC, num_subcores=NS)

_sc_params = pltpu.CompilerParams()
if "needs_layout_passes" in pltpu.CompilerParams.__dataclass_fields__:
    _sc_params = dataclasses.replace(_sc_params, needs_layout_passes=False)
_sc_params_untiled = dataclasses.replace(_sc_params, use_tc_tiling_on_sc=False)


def _pad_agg(src, dst, w, t_chunks):
    e_pad = NW * t_chunks * CHA
    n = src.shape[0]
    ar = jnp.arange(e_pad - n, dtype=jnp.int32)
    srcp = jnp.concatenate([src, ar % N_PROT])
    dstp = jnp.concatenate([dst, N_PROT + ar % TRASH])
    if w is None:
        return srcp, dstp, None
    wp = jnp.concatenate([w, jnp.zeros((e_pad - n,), jnp.float32)])
    return srcp, dstp, wp


def _pack_dec(src, dst, t_chunks):
    e_pad = NW * t_chunks * CHD
    n = src.shape[0]
    ar = jnp.arange(e_pad - n, dtype=jnp.int32)
    srcp = jnp.concatenate([src, ar % N_PROT])
    dstp = jnp.concatenate([dst, ar % N_DIS])
    return jnp.stack([srcp.reshape(NW, t_chunks, CHD),
                      dstp.reshape(NW, t_chunks, CHD)], axis=2)


def _fill_zeros(buf_v):
    z16 = jnp.zeros((16,), jnp.float32)

    @pl.loop(0, RB)
    def _(i):
        for j in range(D // 16):
            buf_v[i, pl.ds(j * 16, 16)] = z16


def _zero_acc(buf_v, acc_sh, s):
    @pl.loop(s, N_PROT // RB, step=NS)
    def _(t):
        pltpu.sync_copy(buf_v.at[pl.ds(0, RB)], acc_sh.at[pl.ds(t * RB, RB)])


def _write_acc(acc_sh, out_hbm, c, s):
    @pl.loop(s, N_PROT // RB, step=NS)
    def _(t):
        pltpu.sync_copy(acc_sh.at[pl.ds(t * RB, RB)],
                        out_hbm.at[c].at[pl.ds(t * RB, RB)])


def _seg_phase(xp_hbm, src_hbm, dst_hbm, w_hbm, acc_sh, wid,
               sb, db, wb, rows, ssb, sdb, sg, ss, t_chunks, weighted):
    base0 = wid * t_chunks * CHA

    def issue_sb(t, b):
        pltpu.async_copy(src_hbm.at[pl.ds(base0 + t * CHA, CHA)], sb[b],
                         ssb[b])
        if weighted:
            pltpu.async_copy(w_hbm.at[pl.ds(base0 + t * CHA, CHA)], wb[b],
                             ssb[b])

    def wait_sb(t, b):
        pltpu.make_async_copy(src_hbm.at[pl.ds(base0 + t * CHA, CHA)], sb[b],
                              ssb[b]).wait()
        if weighted:
            pltpu.make_async_copy(w_hbm.at[pl.ds(base0 + t * CHA, CHA)],
                                  wb[b], ssb[b]).wait()

    def issue_db(t, b):
        pltpu.async_copy(dst_hbm.at[pl.ds(base0 + t * CHA, CHA)], db[b],
                         sdb[b])

    def wait_db(t, b):
        pltpu.make_async_copy(dst_hbm.at[pl.ds(base0 + t * CHA, CHA)], db[b],
                              sdb[b]).wait()

    def issue_g(t, b):
        pltpu.async_copy(xp_hbm.at[sb[b]], rows[b], sg[b])

    def wait_g(t, b):
        pltpu.make_async_copy(xp_hbm.at[sb[b]], rows[b], sg[b]).wait()

    def issue_s(t, b):
        pltpu.async_copy(rows[b], acc_sh.at[db[b]], ss[b], add=True)

    def wait_s(t, b):
        pltpu.make_async_copy(rows[b], acc_sh.at[db[b]], ss[b]).wait()

    def mult(t, b):
        if weighted:
            @plsc.parallel_loop(0, CHA, unroll=4)
            def _(e):
                wv = plsc.load_gather(wb[b], [jnp.full((16,), e, jnp.int32)])
                rbuf = rows[b]
                for j in range(D // 16):
                    sl = pl.ds(j * 16, 16)
                    rbuf[e, sl] = rbuf[e, sl] * wv

    def body(c, b, do_ws, do_next, do_sb2):
        b1, b2 = (b + 1) % 3, (b + 2) % 3
        if do_ws:
            wait_s(c - 2, b1)
        if do_next:
            issue_db(c + 1, b1)
            wait_sb(c + 1, b1)
            issue_g(c + 1, b1)
        wait_g(c, b)
        if do_sb2:
            issue_sb(c + 2, b2)
        mult(c, b)
        wait_db(c, b)
        issue_s(c, b)

    issue_sb(0, 0)
    issue_sb(1, 1)
    issue_db(0, 0)
    wait_sb(0, 0)
    issue_g(0, 0)

    body(0, 0, False, True, True)
    body(1, 1, False, True, True)

    @pl.loop(0, (t_chunks - 4) // 3)
    def _(tau):
        for i in range(3):
            c = 2 + tau * 3 + i
            body(c, (2 + i) % 3, True, True, True)

    body(t_chunks - 2, (t_chunks - 2) % 3, True, True, False)
    body(t_chunks - 1, (t_chunks - 1) % 3, True, False, False)
    wait_s(t_chunks - 2, (t_chunks - 2) % 3)
    wait_s(t_chunks - 1, (t_chunks - 1) % 3)


def _agg_body(xp_hbm, srcpp_hbm, dstpp_hbm, wpp_hbm, srcpd_hbm, dstpd_hbm,
              aggpp_hbm, aggpd_hbm,
              sb0, sb1, sb2, db0, db1, db2, wb0, wb1, wb2,
              rows0, rows1, rows2, acc_sh,
              ssb0, ssb1, ssb2, sdb0, sdb1, sdb2,
              sg0, sg1, sg2, ss0, ss1, ss2):
    c = lax.axis_index("c")
    s = lax.axis_index("s")
    wid = s * NC + c
    sb = [sb0, sb1, sb2]
    db = [db0, db1, db2]
    wb = [wb0, wb1, wb2]
    rows = [rows0, rows1, rows2]
    ssb = [ssb0, ssb1, ssb2]
    sdb = [sdb0, sdb1, sdb2]
    sg = [sg0, sg1, sg2]
    ss = [ss0, ss1, ss2]

    _fill_zeros(rows0)
    _zero_acc(rows0, acc_sh, s)
    plsc.subcore_barrier()

    _seg_phase(xp_hbm, srcpp_hbm, dstpp_hbm, wpp_hbm, acc_sh, wid,
               sb, db, wb, rows, ssb, sdb, sg, ss, T_PP, True)
    plsc.subcore_barrier()
    _write_acc(acc_sh, aggpp_hbm, c, s)
    _fill_zeros(rows0)
    _zero_acc(rows0, acc_sh, s)
    plsc.subcore_barrier()

    _seg_phase(xp_hbm, srcpd_hbm, dstpd_hbm, None, acc_sh, wid,
               sb, db, wb, rows, ssb, sdb, sg, ss, T_PD, False)
    plsc.subcore_barrier()
    _write_acc(acc_sh, aggpd_hbm, c, s)


@jax.jit
def _aggregate(xp, srcpp, dstpp, wpp, srcpd, dstpd):
    f = pl.kernel(
        _agg_body,
        out_type=(jax.ShapeDtypeStruct((NC, N_PROT, D), jnp.float32),
                  jax.ShapeDtypeStruct((NC, N_DIS, D), jnp.float32)),
        mesh=_mesh,
        scratch_types=[
            pltpu.VMEM((CHA,), jnp.int32),
            pltpu.VMEM((CHA,), jnp.int32),
            pltpu.VMEM((CHA,), jnp.int32),
            pltpu.VMEM((CHA,), jnp.int32),
            pltpu.VMEM((CHA,), jnp.int32),
            pltpu.VMEM((CHA,), jnp.int32),
            pltpu.VMEM((CHA,), jnp.float32),
            pltpu.VMEM((CHA,), jnp.float32),
            pltpu.VMEM((CHA,), jnp.float32),
            pltpu.VMEM((CHA, D), jnp.float32),
            pltpu.VMEM((CHA, D), jnp.float32),
            pltpu.VMEM((CHA, D), jnp.float32),
            pltpu.VMEM_SHARED((ACC_N, D), jnp.float32),
            pltpu.SemaphoreType.DMA,
            pltpu.SemaphoreType.DMA,
            pltpu.SemaphoreType.DMA,
            pltpu.SemaphoreType.DMA,
            pltpu.SemaphoreType.DMA,
            pltpu.SemaphoreType.DMA,
            pltpu.SemaphoreType.DMA,
            pltpu.SemaphoreType.DMA,
            pltpu.SemaphoreType.DMA,
            pltpu.SemaphoreType.DMA,
            pltpu.SemaphoreType.DMA,
            pltpu.SemaphoreType.DMA,
        ],
        compiler_params=_sc_params,
    )
    return f(xp, srcpp, dstpp, wpp, srcpd, dstpd)


def _enc_block(x_ref, a_ref, ws_ref, wn_ref, o_ref):
    agg = a_ref[0] + a_ref[1]
    h = jnp.maximum(
        jnp.dot(x_ref[...], ws_ref[0], preferred_element_type=jnp.float32)
        + jnp.dot(agg, wn_ref[0], preferred_element_type=jnp.float32),
        0.0)
    hb = h.astype(jnp.bfloat16)
    lo = lax.bitcast_convert_type(hb[:, :D // 2], jnp.uint16)
    hi = lax.bitcast_convert_type(hb[:, D // 2:], jnp.uint16)
    packed = (lo.astype(jnp.uint32)
              | (hi.astype(jnp.uint32) << 16)).astype(jnp.int32)
    o_ref[...] = packed


@jax.jit
def _encode(x, agg2, w_self, w_nbr):
    n = x.shape[0]
    br = 2000
    return pl.pallas_call(
        _enc_block,
        grid=(n // br,),
        in_specs=[
            pl.BlockSpec((br, D), lambda i: (i, 0)),
            pl.BlockSpec((NC, br, D), lambda i: (0, i, 0)),
            pl.BlockSpec((1, D, D), lambda i: (0, 0, 0)),
            pl.BlockSpec((1, D, D), lambda i: (0, 0, 0)),
        ],
        out_specs=pl.BlockSpec((br, D // 2), lambda i: (i, 0)),
        out_shape=jax.ShapeDtypeStruct((n, D // 2), jnp.int32),
    )(x, agg2, w_self[None], w_nbr[None])


def _decode_body(hp_hbm, hd_hbm, sup_hbm, out_hbm,
                 idx_v, l0, l1, l2, r0, r1, r2, oall_v,
                 semi, sl0, sl1, sl2, sr0, sr1, sr2):
    c = lax.axis_index("c")
    s = lax.axis_index("s")
    wid = s * NC + c
    lbufs = [l0, l1, l2]
    rbufs = [r0, r1, r2]
    sls = [sl0, sl1, sl2]
    srs = [sr0, sr1, sr2]
    lane = lax.iota(jnp.int32, 16)

    pltpu.async_copy(sup_hbm.at[wid], idx_v, semi).wait()

    def issue(t, b):
        pltpu.async_copy(hp_hbm.at[idx_v.at[t, 0]], lbufs[b], sls[b])
        pltpu.async_copy(hd_hbm.at[idx_v.at[t, 1]], rbufs[b], srs[b])

    def wait(t, b):
        pltpu.make_async_copy(hp_hbm.at[idx_v.at[t, 0]], lbufs[b],
                              sls[b]).wait()
        pltpu.make_async_copy(hd_hbm.at[idx_v.at[t, 1]], rbufs[b],
                              srs[b]).wait()

    def compute(t, b):
        l_v, r_v = lbufs[b], rbufs[b]

        @plsc.parallel_loop(0, CHD // 16, unroll=2)
        def _(g):
            out16 = jnp.zeros((16,), jnp.float32)
            for r in range(16):
                e = g * 16 + r
                acc = None
                for j in range(D // 32):
                    sl = pl.ds(j * 16, 16)
                    lv = plsc.bitcast(l_v[e, sl], jnp.bfloat16)
                    rv = plsc.bitcast(r_v[e, sl], jnp.bfloat16)
                    prod = lv * rv
                    acc = prod if acc is None else acc + prod
                pa, pb = plsc.unpack(acc, format=plsc.PackFormat.INTERLEAVED)
                dot = jnp.sum(pa + pb)
                out16 = jnp.where(lane == r, dot, out16)
            oall_v[t, pl.ds(g * 16, 16)] = out16

    issue(0, 0)
    issue(1, 1)

    @pl.loop(0, (T_SUP - 2) // 3)
    def _(tau):
        for i in range(3):
            t = tau * 3 + i
            issue(t + 2, (i + 2) % 3)
            wait(t, i)
            compute(t, i)

    for t in (T_SUP - 2, T_SUP - 1):
        wait(t, t % 3)
        compute(t, t % 3)

    pltpu.sync_copy(oall_v, out_hbm.at[wid])


@jax.jit
def _decode(hp, hd, sup_pack):
    f = pl.kernel(
        _decode_body,
        out_type=jax.ShapeDtypeStruct((NW, T_SUP, CHD), jnp.float32),
        mesh=_mesh,
        scratch_types=[
            pltpu.VMEM((T_SUP, 2, CHD), jnp.int32),
            pltpu.VMEM((CHD, D // 2), jnp.int32),
            pltpu.VMEM((CHD, D // 2), jnp.int32),
            pltpu.VMEM((CHD, D // 2), jnp.int32),
            pltpu.VMEM((CHD, D // 2), jnp.int32),
            pltpu.VMEM((CHD, D // 2), jnp.int32),
            pltpu.VMEM((CHD, D // 2), jnp.int32),
            pltpu.VMEM((T_SUP, CHD), jnp.float32),
            pltpu.SemaphoreType.DMA,
            pltpu.SemaphoreType.DMA,
            pltpu.SemaphoreType.DMA,
            pltpu.SemaphoreType.DMA,
            pltpu.SemaphoreType.DMA,
            pltpu.SemaphoreType.DMA,
            pltpu.SemaphoreType.DMA,
        ],
        compiler_params=_sc_params_untiled,
    )
    return f(hp, hd, sup_pack)


def kernel(x_protein, x_disease, edge_index_pp, edge_attr_pp, edge_index_pd,
           sup_edge_index, W_self_p, W_nbr_pp, W_self_d, W_nbr_pd):
    srcpp, dstpp, wpp = _pad_agg(edge_index_pp[0], edge_index_pp[1],
                                 edge_attr_pp[:, 0], T_PP)
    srcpd, dstpd, _ = _pad_agg(edge_index_pd[0], edge_index_pd[1], None,
                               T_PD)
    sup_pack = _pack_dec(sup_edge_index[0], sup_edge_index[1], T_SUP)

    aggpp2, aggpd2 = _aggregate(x_protein, srcpp, dstpp, wpp, srcpd, dstpd)
    hp32 = _encode(x_protein, aggpp2, W_self_p, W_nbr_pp)
    hd32 = _encode(x_disease, aggpd2, W_self_d, W_nbr_pd)
    scores = _decode(hp32, hd32, sup_pack)
    return scores.reshape(-1)[:E_SUP]
